# baseline (speedup 1.0000x reference)
import functools

import jax
import jax.numpy as jnp
from jax import lax
from jax.experimental import pallas as pl
from jax.experimental.pallas import tpu as pltpu
from jax.experimental.pallas import tpu_sc as plsc

N_NODES = 100000
NUM_ADJ = 64
BATCH = 1024

IDX10 = (47, 9, 2, 38, 42, 63, 46, 5, 14, 7)
IDX25A = (62, 30, 57, 43, 35, 44, 42, 3, 22, 20, 19, 6, 63, 26, 41, 17, 40,
          8, 45, 36, 27, 53, 39, 34, 25)
IDX25B = (25, 28, 34, 2, 37, 57, 44, 40, 47, 31, 30, 63, 58, 20, 27, 29, 42,
          5, 22, 17, 4, 1, 41, 32, 16)

NW = 32
SEEDS_W = BATCH // NW
FRONT_W = SEEDS_W * len(IDX10)
A_W = SEEDS_W * 25
C_W = FRONT_W * 25


def _chunks(total, size=128):
    out, start = [], 0
    while start < total:
        out.append((start, min(size, total - start)))
        start += size
    return out


def _tidx(ids, col):
    cj = (col // 8) * (NTILE_PAD * 1024) + (col % 8) * 128
    return (((ids >> jnp.int32(7)) << jnp.int32(10))
            + (ids & jnp.int32(127)) + jnp.int32(cj))


def _sampler(inputs_hbm, table_hbm, a_hbm, b_hbm, c_hbm,
             seed_v, idxa_v, idxb_v, idxc_v, outa_v, outb_v, outc_v, sem):
    wid = lax.axis_index("s") * 2 + lax.axis_index("c")
    base = wid * SEEDS_W

    pltpu.sync_copy(inputs_hbm.at[pl.ds(base, SEEDS_W)], seed_v)

    iota = lax.iota(jnp.int32, 16)

    for h in range(SEEDS_W // 16):
        rvec = iota + jnp.int32(16 * h)
        ids = seed_v[pl.ds(16 * h, 16)]
        for j, col in enumerate(IDX25A):
            plsc.store_scatter(idxa_v, [rvec * jnp.int32(25) + jnp.int32(j)],
                               _tidx(ids, col))
        for j, col in enumerate(IDX10):
            plsc.store_scatter(idxb_v, [rvec * jnp.int32(10) + jnp.int32(j)],
                               _tidx(ids, col))

    copies = []
    for start, size in _chunks(A_W):
        copies.append(pltpu.async_copy(
            table_hbm.at[idxa_v.at[pl.ds(start, size)]],
            outa_v.at[pl.ds(start, size)], sem))
    for start, size in _chunks(FRONT_W):
        copies.append(pltpu.async_copy(
            table_hbm.at[idxb_v.at[pl.ds(start, size)]],
            outb_v.at[pl.ds(start, size)], sem))
    for cp in copies:
        cp.wait()

    def chunk_body(ci, carry):
        rvec = ci * jnp.int32(16) + iota
        ids = plsc.bitcast(outb_v[pl.ds(ci * 16, 16)], jnp.int32)
        for j, col in enumerate(IDX25B):
            plsc.store_scatter(idxc_v, [rvec * jnp.int32(25) + jnp.int32(j)],
                               _tidx(ids, col))
        return carry
    lax.fori_loop(jnp.int32(0), jnp.int32(FRONT_W // 16), chunk_body,
                  jnp.int32(0))

    copies = []
    for start, size in _chunks(C_W):
        copies.append(pltpu.async_copy(
            table_hbm.at[idxc_v.at[pl.ds(start, size)]],
            outc_v.at[pl.ds(start, size)], sem))
    for cp in copies:
        cp.wait()

    pltpu.sync_copy(outa_v, a_hbm.at[pl.ds(wid * A_W, A_W)])
    pltpu.sync_copy(outb_v, b_hbm.at[pl.ds(wid * FRONT_W, FRONT_W)])
    pltpu.sync_copy(outc_v, c_hbm.at[pl.ds(wid * C_W, C_W)])


NTILE = N_NODES // 128
NMAIN = NTILE * 128
NTAIL = N_NODES - NMAIN
NTILE_PAD = NTILE + 1
NROWT = 8 * NTILE_PAD
DCHUNK = 49 * 128


def _detile(table2d_hbm, tail_hbm, flat_hbm, buf0, buf1, tbuf, si0, si1,
            so0, so1, st):
    wid = lax.axis_index("s") * 2 + lax.axis_index("c")
    g = wid // jnp.int32(4)
    q = wid % jnp.int32(4)
    row8 = pl.multiple_of(g * jnp.int32(8), 8)
    base = jnp.where(q < jnp.int32(3), q * jnp.int32(195), jnp.int32(585))
    bufs, sin, sout = [buf0, buf1], [si0, si1], [so0, so1]

    tail_in = []
    for r in range(8):
        j = g * jnp.int32(8) + jnp.int32(r)
        tail_in.append(pltpu.async_copy(
            tail_hbm.at[pl.ds(j * jnp.int32(NTAIL), NTAIL)],
            tbuf.at[jnp.int32(r), pl.ds(0, NTAIL)], st))

    def step_off(t):
        return pl.multiple_of((base + jnp.int32(t * 49)) * jnp.int32(128),
                              128)

    def src(t):
        return table2d_hbm.at[pl.ds(row8, 8), pl.ds(step_off(t), DCHUNK)]

    def write_out(t, b):
        trow0 = g * jnp.int32(NTILE_PAD) + base + jnp.int32(t * 49)
        return [pltpu.async_copy(
            bufs[b].at[pl.ds(0, 8), pl.ds(c * 128, 128)],
            flat_hbm.at[trow0 + jnp.int32(c)],
            sout[b]) for c in range(49)]

    in_c, out_c = {}, {}
    in_c[0] = pltpu.async_copy(src(0), bufs[0], sin[0])
    in_c[1] = pltpu.async_copy(src(1), bufs[1], sin[1])
    for t in range(4):
        in_c[t].wait()
        out_c[t] = write_out(t, t % 2)
        if t + 2 < 4:
            for cp in out_c[t]:
                cp.wait()
            in_c[t + 2] = pltpu.async_copy(src(t + 2), bufs[t % 2], sin[t % 2])
    for t in (2, 3):
        for cp in out_c[t]:
            cp.wait()
    for cp in tail_in:
        cp.wait()
    pltpu.sync_copy(
        tbuf, flat_hbm.at[g * jnp.int32(NTILE_PAD) + jnp.int32(NTILE)])


@jax.jit
def _run(inputs32, table2d, tail32):
    mesh = plsc.VectorSubcoreMesh(core_axis_name="c", subcore_axis_name="s")
    flat3d = pl.kernel(
        _detile, mesh=mesh,
        compiler_params=pltpu.CompilerParams(needs_layout_passes=False),
        out_type=jax.ShapeDtypeStruct((NROWT, 8, 128), jnp.uint32),
        scratch_types=[
            pltpu.VMEM((8, DCHUNK), jnp.uint32),
            pltpu.VMEM((8, DCHUNK), jnp.uint32),
            pltpu.VMEM((8, 128), jnp.uint32),
            pltpu.SemaphoreType.DMA,
            pltpu.SemaphoreType.DMA,
            pltpu.SemaphoreType.DMA,
            pltpu.SemaphoreType.DMA,
            pltpu.SemaphoreType.DMA,
        ],
    )(table2d, tail32)
    flat_table = flat3d.reshape(-1)
    fn = functools.partial(
        pl.kernel, mesh=mesh,
        compiler_params=pltpu.CompilerParams(needs_layout_passes=False),
        out_type=[
            jax.ShapeDtypeStruct((BATCH * 25,), jnp.uint32),
            jax.ShapeDtypeStruct((BATCH * 10,), jnp.uint32),
            jax.ShapeDtypeStruct((BATCH * 250,), jnp.uint32),
        ],
        scratch_types=[
            pltpu.VMEM((SEEDS_W,), jnp.int32),
            pltpu.VMEM((A_W,), jnp.int32),
            pltpu.VMEM((FRONT_W,), jnp.int32),
            pltpu.VMEM((C_W,), jnp.int32),
            pltpu.VMEM((A_W,), jnp.uint32),
            pltpu.VMEM((FRONT_W,), jnp.uint32),
            pltpu.VMEM((C_W,), jnp.uint32),
            pltpu.SemaphoreType.DMA,
        ],
    )(_sampler)
    return fn(inputs32, flat_table)


def kernel(inputs, adj_info):
    inputs32 = inputs.astype(jnp.int32)
    table2d = jnp.zeros((NUM_ADJ, N_NODES), jnp.uint32)
    tail32 = adj_info[N_NODES - NTAIL:, :].astype(jnp.uint32).T.reshape(-1)
    a32, b32, c32 = _run(inputs32, table2d, tail32)
    a = a32.astype(jnp.int64).reshape(BATCH, 25)
    b = b32.astype(jnp.int64).reshape(BATCH, 10)
    c = c32.astype(jnp.int64).reshape(BATCH, 10, 25)
    return (inputs, a, b, c)


# speedup vs baseline: 1.1214x; 1.1214x over previous
import functools

import jax
import jax.numpy as jnp
from jax import lax
from jax.experimental import pallas as pl
from jax.experimental.pallas import tpu as pltpu
from jax.experimental.pallas import tpu_sc as plsc

N_NODES = 100000
NUM_ADJ = 64
BATCH = 1024

IDX10 = (47, 9, 2, 38, 42, 63, 46, 5, 14, 7)
IDX25A = (62, 30, 57, 43, 35, 44, 42, 3, 22, 20, 19, 6, 63, 26, 41, 17, 40,
          8, 45, 36, 27, 53, 39, 34, 25)
IDX25B = (25, 28, 34, 2, 37, 57, 44, 40, 47, 31, 30, 63, 58, 20, 27, 29, 42,
          5, 22, 17, 4, 1, 41, 32, 16)

NW = 32
SEEDS_W = BATCH // NW
FRONT_W = SEEDS_W * len(IDX10)
A_W = SEEDS_W * 25
C_W = FRONT_W * 25


def _chunks(total, size=128):
    out, start = [], 0
    while start < total:
        out.append((start, min(size, total - start)))
        start += size
    return out


def _tidx(ids, col):
    cj = (col // 8) * (NTILE_PAD * 1024) + (col % 8) * 128
    return (((ids >> jnp.int32(7)) << jnp.int32(10))
            + (ids & jnp.int32(127)) + jnp.int32(cj))


def _sampler(inputs_hbm, table_hbm, a_hbm, b_hbm, c_hbm,
             seed_v, idxa_v, idxb_v, idxc_v, outa_v, outb_v, outc_v, sem):
    wid = lax.axis_index("s") * 2 + lax.axis_index("c")
    base = wid * SEEDS_W

    pltpu.sync_copy(inputs_hbm.at[pl.ds(base, SEEDS_W)], seed_v)

    iota = lax.iota(jnp.int32, 16)

    for h in range(SEEDS_W // 16):
        rvec = iota + jnp.int32(16 * h)
        ids = seed_v[pl.ds(16 * h, 16)]
        for j, col in enumerate(IDX25A):
            plsc.store_scatter(idxa_v, [rvec * jnp.int32(25) + jnp.int32(j)],
                               _tidx(ids, col))
        for j, col in enumerate(IDX10):
            plsc.store_scatter(idxb_v, [rvec * jnp.int32(10) + jnp.int32(j)],
                               _tidx(ids, col))

    copies = []
    for start, size in _chunks(A_W):
        copies.append(pltpu.async_copy(
            table_hbm.at[idxa_v.at[pl.ds(start, size)]],
            outa_v.at[pl.ds(start, size)], sem))
    for start, size in _chunks(FRONT_W):
        copies.append(pltpu.async_copy(
            table_hbm.at[idxb_v.at[pl.ds(start, size)]],
            outb_v.at[pl.ds(start, size)], sem))
    for cp in copies:
        cp.wait()

    def chunk_body(ci, carry):
        rvec = ci * jnp.int32(16) + iota
        ids = plsc.bitcast(outb_v[pl.ds(ci * 16, 16)], jnp.int32)
        for j, col in enumerate(IDX25B):
            plsc.store_scatter(idxc_v, [rvec * jnp.int32(25) + jnp.int32(j)],
                               _tidx(ids, col))
        return carry
    lax.fori_loop(jnp.int32(0), jnp.int32(FRONT_W // 16), chunk_body,
                  jnp.int32(0))

    copies = []
    for start, size in _chunks(C_W):
        copies.append(pltpu.async_copy(
            table_hbm.at[idxc_v.at[pl.ds(start, size)]],
            outc_v.at[pl.ds(start, size)], sem))
    for cp in copies:
        cp.wait()

    pltpu.sync_copy(outa_v, a_hbm.at[pl.ds(wid * A_W, A_W)])
    pltpu.sync_copy(outb_v, b_hbm.at[pl.ds(wid * FRONT_W, FRONT_W)])
    pltpu.sync_copy(outc_v, c_hbm.at[pl.ds(wid * C_W, C_W)])


NTILE = N_NODES // 128
NMAIN = NTILE * 128
NTAIL = N_NODES - NMAIN
NTILE_PAD = NTILE + 1
NROWT = 8 * NTILE_PAD
DCHUNK = 49 * 128


def _detile(table2d_hbm, tail_hbm, flat_hbm, buf0, buf1, tbuf, si0, si1,
            so0, so1, st):
    wid = lax.axis_index("s") * 2 + lax.axis_index("c")
    g = wid // jnp.int32(4)
    q = wid % jnp.int32(4)
    row8 = pl.multiple_of(g * jnp.int32(8), 8)
    base = jnp.where(q < jnp.int32(3), q * jnp.int32(195), jnp.int32(585))
    bufs, sin, sout = [buf0, buf1], [si0, si1], [so0, so1]

    tail_in = []
    for r in range(8):
        j = g * jnp.int32(8) + jnp.int32(r)
        tail_in.append(pltpu.async_copy(
            tail_hbm.at[pl.ds(j * jnp.int32(NTAIL), NTAIL)],
            tbuf.at[jnp.int32(r), pl.ds(0, NTAIL)], st))

    def step_off(t):
        return pl.multiple_of((base + jnp.int32(t * 49)) * jnp.int32(128),
                              128)

    def read_in(t, b):
        off = step_off(t)
        return [pltpu.async_copy(
            table2d_hbm.at[pl.ds(row8, 8),
                           pl.ds(off + jnp.int32(c * 128), 128)],
            bufs[b].at[jnp.int32(c)],
            sin[b]) for c in range(49)]

    def write_out(t, b):
        trow0 = g * jnp.int32(NTILE_PAD) + base + jnp.int32(t * 49)
        return pltpu.async_copy(
            bufs[b], flat_hbm.at[pl.ds(trow0, 49)], sout[b])

    in_c, out_c = {}, {}
    in_c[0] = read_in(0, 0)
    in_c[1] = read_in(1, 1)
    for t in range(4):
        for cp in in_c[t]:
            cp.wait()
        out_c[t] = write_out(t, t % 2)
        if t + 2 < 4:
            out_c[t].wait()
            in_c[t + 2] = read_in(t + 2, t % 2)
    for t in (2, 3):
        out_c[t].wait()
    for cp in tail_in:
        cp.wait()
    pltpu.sync_copy(
        tbuf, flat_hbm.at[g * jnp.int32(NTILE_PAD) + jnp.int32(NTILE)])


@jax.jit
def _run(inputs32, table2d, tail32):
    mesh = plsc.VectorSubcoreMesh(core_axis_name="c", subcore_axis_name="s")
    flat3d = pl.kernel(
        _detile, mesh=mesh,
        compiler_params=pltpu.CompilerParams(needs_layout_passes=False),
        out_type=jax.ShapeDtypeStruct((NROWT, 8, 128), jnp.uint32),
        scratch_types=[
            pltpu.VMEM((49, 8, 128), jnp.uint32),
            pltpu.VMEM((49, 8, 128), jnp.uint32),
            pltpu.VMEM((8, 128), jnp.uint32),
            pltpu.SemaphoreType.DMA,
            pltpu.SemaphoreType.DMA,
            pltpu.SemaphoreType.DMA,
            pltpu.SemaphoreType.DMA,
            pltpu.SemaphoreType.DMA,
        ],
    )(table2d, tail32)
    flat_table = flat3d.reshape(-1)
    fn = functools.partial(
        pl.kernel, mesh=mesh,
        compiler_params=pltpu.CompilerParams(needs_layout_passes=False),
        out_type=[
            jax.ShapeDtypeStruct((BATCH * 25,), jnp.uint32),
            jax.ShapeDtypeStruct((BATCH * 10,), jnp.uint32),
            jax.ShapeDtypeStruct((BATCH * 250,), jnp.uint32),
        ],
        scratch_types=[
            pltpu.VMEM((SEEDS_W,), jnp.int32),
            pltpu.VMEM((A_W,), jnp.int32),
            pltpu.VMEM((FRONT_W,), jnp.int32),
            pltpu.VMEM((C_W,), jnp.int32),
            pltpu.VMEM((A_W,), jnp.uint32),
            pltpu.VMEM((FRONT_W,), jnp.uint32),
            pltpu.VMEM((C_W,), jnp.uint32),
            pltpu.SemaphoreType.DMA,
        ],
    )(_sampler)
    return fn(inputs32, flat_table)


def kernel(inputs, adj_info):
    inputs32 = inputs.astype(jnp.int32)
    table2d = adj_info.astype(jnp.uint32).T
    tail32 = adj_info[N_NODES - NTAIL:, :].astype(jnp.uint32).T.reshape(-1)
    a32, b32, c32 = _run(inputs32, table2d, tail32)
    a = a32.astype(jnp.int64).reshape(BATCH, 25)
    b = b32.astype(jnp.int64).reshape(BATCH, 10)
    c = c32.astype(jnp.int64).reshape(BATCH, 10, 25)
    return (inputs, a, b, c)
